# baseline (speedup 1.0000x reference)
import jax
import jax.numpy as jnp
from jax import lax
from jax.experimental import pallas as pl
from jax.experimental.pallas import tpu as pltpu
from jax.experimental.pallas import tpu_sc as plsc

N_NODES = 10000
N_EDGES = 320000
D = 128

NCORE = 2
NSUB = 16
NW = NCORE * NSUB
CHUNK = 64
NPACK = 80
NCHUNK = 2 * NPACK
EPW = NCHUNK * CHUNK
E_PAD = NW * EPW
ROWS_PER_SUB = 624
TAIL_ROWS = N_NODES - NSUB * ROWS_PER_SUB


def _mul_chunk(rows, w_v, kk, half):
    def group_body(g, carry):
        w16 = w_v[kk, pl.ds(half * CHUNK + g * 16, 16)]
        for l in range(16):
            wvec = jnp.full((16,), w16[l], jnp.float32)
            e = g * 16 + l
            for j in range(D // 16):
                sl = pl.ds(j * 16, 16)
                rows[e, sl] = rows[e, sl] * wvec
        return carry

    lax.fori_loop(0, CHUNK // 16, group_body, 0)


def _sc_body(feat_hbm, src_hbm, dst_hbm, ew_hbm, out_hbm,
             src_v, dst_v, w_v, rows0, rows1, acc_sh,
             gsem0, gsem1, ssem0, ssem1):
    c = lax.axis_index("c")
    s = lax.axis_index("s")
    wid = c * NSUB + s

    pltpu.sync_copy(src_hbm.at[wid], src_v)
    pltpu.sync_copy(dst_hbm.at[wid], dst_v)
    pltpu.sync_copy(ew_hbm.at[wid], w_v)

    def zrow(i, carry):
        for j in range(D // 16):
            rows0[i, pl.ds(j * 16, 16)] = jnp.zeros((16,), jnp.float32)
        return carry

    lax.fori_loop(0, CHUNK, zrow, 0)

    base = s * ROWS_PER_SUB
    nfull = ROWS_PER_SUB // CHUNK
    rem = ROWS_PER_SUB - nfull * CHUNK
    for k in range(nfull):
        pltpu.sync_copy(rows0, acc_sh.at[pl.ds(base + k * CHUNK, CHUNK)])
    if rem:
        pltpu.sync_copy(rows0.at[pl.ds(0, rem)],
                        acc_sh.at[pl.ds(base + nfull * CHUNK, rem)])

    @pl.when(s == NSUB - 1)
    def _zero_tail():
        pltpu.sync_copy(rows0.at[pl.ds(0, TAIL_ROWS)],
                        acc_sh.at[pl.ds(NSUB * ROWS_PER_SUB, TAIL_ROWS)])

    plsc.subcore_barrier()

    bufs = (rows0, rows1)
    gsems = (gsem0, gsem1)
    ssems = (ssem0, ssem1)

    def src_at(kk, half):
        return src_v.at[kk, pl.ds(half * CHUNK, CHUNK)]

    def dst_at(kk, half):
        return dst_v.at[kk, pl.ds(half * CHUNK, CHUNK)]

    pltpu.async_copy(feat_hbm.at[src_at(0, 0)], rows0, gsem0)

    def pair_body(kk, carry):
        for b in range(2):
            k = kk * 2 + b
            buf, gsem, ssem = bufs[b], gsems[b], ssems[b]
            nbuf, ngsem, nssem = bufs[1 - b], gsems[1 - b], ssems[1 - b]
            kk_next, h_next = (kk, 1) if b == 0 else (kk + 1, 0)
            kk_prev, h_prev = (kk, 0) if b == 1 else (kk - 1, 1)

            pltpu.make_async_copy(
                feat_hbm.at[src_at(kk, b)], buf, gsem).wait()

            @pl.when(k + 1 < NCHUNK)
            def _prefetch():
                @pl.when(k >= 1)
                def _drain():
                    pltpu.make_async_copy(
                        nbuf, acc_sh.at[dst_at(kk_prev, h_prev)],
                        nssem).wait()

                pltpu.async_copy(
                    feat_hbm.at[src_at(kk_next, h_next)], nbuf, ngsem)

            _mul_chunk(buf, w_v, kk, b)

            pltpu.async_copy(buf, acc_sh.at[dst_at(kk, b)], ssem, add=True)
        return carry

    lax.fori_loop(0, NCHUNK // 2, pair_body, 0)

    pltpu.make_async_copy(
        rows0, acc_sh.at[dst_at(NPACK - 1, 0)], ssem0).wait()
    pltpu.make_async_copy(
        rows1, acc_sh.at[dst_at(NPACK - 1, 1)], ssem1).wait()
    plsc.subcore_barrier()

    pltpu.sync_copy(acc_sh.at[pl.ds(base, ROWS_PER_SUB)],
                    out_hbm.at[c, pl.ds(base, ROWS_PER_SUB)])

    @pl.when(s == NSUB - 1)
    def _flush_tail():
        pltpu.sync_copy(acc_sh.at[pl.ds(NSUB * ROWS_PER_SUB, TAIL_ROWS)],
                        out_hbm.at[c, pl.ds(NSUB * ROWS_PER_SUB, TAIL_ROWS)])


_sc_aggregate = pl.kernel(
    _sc_body,
    out_type=jax.ShapeDtypeStruct((NCORE, N_NODES, D), jnp.float32),
    mesh=plsc.VectorSubcoreMesh(core_axis_name="c", subcore_axis_name="s"),
    scratch_types=[
        pltpu.VMEM((NPACK, 2 * CHUNK), jnp.int32),
        pltpu.VMEM((NPACK, 2 * CHUNK), jnp.int32),
        pltpu.VMEM((NPACK, 2 * CHUNK), jnp.float32),
        pltpu.VMEM((CHUNK, D), jnp.float32),
        pltpu.VMEM((CHUNK, D), jnp.float32),
        pltpu.VMEM_SHARED((N_NODES, D), jnp.float32),
        pltpu.SemaphoreType.DMA,
        pltpu.SemaphoreType.DMA,
        pltpu.SemaphoreType.DMA,
        pltpu.SemaphoreType.DMA,
    ],
)

ROW_BLK = 1000


def _tc_body(p_ref, w_ref, o_ref):
    acc = p_ref[0] + p_ref[1]
    o_ref[...] = jnp.maximum(
        jnp.dot(acc, w_ref[...], preferred_element_type=jnp.float32), 0.0)


def _tc_finish(partials, W):
    return pl.pallas_call(
        _tc_body,
        grid=(N_NODES // ROW_BLK,),
        in_specs=[
            pl.BlockSpec((NCORE, ROW_BLK, D), lambda i: (0, i, 0)),
            pl.BlockSpec((D, D), lambda i: (0, 0)),
        ],
        out_specs=pl.BlockSpec((ROW_BLK, D), lambda i: (i, 0)),
        out_shape=jax.ShapeDtypeStruct((N_NODES, D), jnp.float32),
    )(partials, W)


@jax.jit
def kernel(feat, edge_index, edge_weight, W):
    pad = E_PAD - N_EDGES
    src = jnp.concatenate(
        [edge_index[0], jnp.zeros((pad,), jnp.int32)]
    ).reshape(NW, NPACK, 2 * CHUNK)
    dst = jnp.concatenate(
        [edge_index[1], jnp.zeros((pad,), jnp.int32)]
    ).reshape(NW, NPACK, 2 * CHUNK)
    ew = jnp.concatenate(
        [edge_weight, jnp.zeros((pad,), jnp.float32)]
    ).reshape(NW, NPACK, 2 * CHUNK)
    partials = _sc_aggregate(feat, src, dst, ew)
    return _tc_finish(partials, W)


# speedup vs baseline: 2.8316x; 2.8316x over previous
import jax
import jax.numpy as jnp
from jax import lax
from jax.experimental import pallas as pl
from jax.experimental.pallas import tpu as pltpu
from jax.experimental.pallas import tpu_sc as plsc

N_NODES = 10000
N_EDGES = 320000
D = 128

NCORE = 2
NSUB = 16
NW = NCORE * NSUB
CHUNK = 64
NPACK = 80
NCHUNK = 2 * NPACK
EPW = NCHUNK * CHUNK
E_PAD = NW * EPW
ROWS_PER_SUB = 624
TAIL_ROWS = N_NODES - NSUB * ROWS_PER_SUB


def _mul_chunk(rows, w_v, kk, half):
    def group_body(g, carry):
        w16 = w_v[kk, pl.ds(half * CHUNK + g * 16, 16)]
        for l in range(16):
            wvec = jnp.full((16,), w16[l], jnp.float32)
            e = g * 16 + l
            for j in range(D // 16):
                sl = pl.ds(j * 16, 16)
                rows[e, sl] = rows[e, sl] * wvec
        return carry

    lax.fori_loop(0, CHUNK // 16, group_body, 0)


def _sc_body(feat_hbm, src_hbm, dst_hbm, ew_hbm, out0_hbm, out1_hbm,
             src_v, dst_v, w_v, rows0, rows1, acc_sh,
             gsem0, gsem1, ssem0, ssem1):
    c = lax.axis_index("c")
    s = lax.axis_index("s")
    wid = c * NSUB + s

    pltpu.sync_copy(src_hbm.at[wid], src_v)
    pltpu.sync_copy(dst_hbm.at[wid], dst_v)
    pltpu.sync_copy(ew_hbm.at[wid], w_v)

    def zrow(i, carry):
        for j in range(D // 16):
            rows0[i, pl.ds(j * 16, 16)] = jnp.zeros((16,), jnp.float32)
        return carry

    lax.fori_loop(0, CHUNK, zrow, 0)

    base = s * ROWS_PER_SUB
    nfull = ROWS_PER_SUB // CHUNK
    rem = ROWS_PER_SUB - nfull * CHUNK
    for k in range(nfull):
        pltpu.sync_copy(rows0, acc_sh.at[pl.ds(base + k * CHUNK, CHUNK)])
    if rem:
        pltpu.sync_copy(rows0.at[pl.ds(0, rem)],
                        acc_sh.at[pl.ds(base + nfull * CHUNK, rem)])

    @pl.when(s == NSUB - 1)
    def _zero_tail():
        pltpu.sync_copy(rows0.at[pl.ds(0, TAIL_ROWS)],
                        acc_sh.at[pl.ds(NSUB * ROWS_PER_SUB, TAIL_ROWS)])

    plsc.subcore_barrier()

    bufs = (rows0, rows1)
    gsems = (gsem0, gsem1)
    ssems = (ssem0, ssem1)

    def src_at(kk, half):
        return src_v.at[kk, pl.ds(half * CHUNK, CHUNK)]

    def dst_at(kk, half):
        return dst_v.at[kk, pl.ds(half * CHUNK, CHUNK)]

    pltpu.async_copy(feat_hbm.at[src_at(0, 0)], rows0, gsem0)

    def pair_body(kk, carry):
        for b in range(2):
            k = kk * 2 + b
            buf, gsem, ssem = bufs[b], gsems[b], ssems[b]
            nbuf, ngsem, nssem = bufs[1 - b], gsems[1 - b], ssems[1 - b]
            kk_next, h_next = (kk, 1) if b == 0 else (kk + 1, 0)
            kk_prev, h_prev = (kk, 0) if b == 1 else (kk - 1, 1)

            pltpu.make_async_copy(
                feat_hbm.at[src_at(kk, b)], buf, gsem).wait()

            @pl.when(k + 1 < NCHUNK)
            def _prefetch():
                @pl.when(k >= 1)
                def _drain():
                    pltpu.make_async_copy(
                        nbuf, acc_sh.at[dst_at(kk_prev, h_prev)],
                        nssem).wait()

                pltpu.async_copy(
                    feat_hbm.at[src_at(kk_next, h_next)], nbuf, ngsem)

            _mul_chunk(buf, w_v, kk, b)

            pltpu.async_copy(buf, acc_sh.at[dst_at(kk, b)], ssem, add=True)
        return carry

    lax.fori_loop(0, NCHUNK // 2, pair_body, 0)

    pltpu.make_async_copy(
        rows0, acc_sh.at[dst_at(NPACK - 1, 0)], ssem0).wait()
    pltpu.make_async_copy(
        rows1, acc_sh.at[dst_at(NPACK - 1, 1)], ssem1).wait()
    plsc.subcore_barrier()

    for cc, out_hbm in ((0, out0_hbm), (1, out1_hbm)):
        @pl.when(c == cc)
        def _flush(out_hbm=out_hbm):
            pltpu.sync_copy(acc_sh.at[pl.ds(base, ROWS_PER_SUB)],
                            out_hbm.at[pl.ds(base, ROWS_PER_SUB)])

            @pl.when(s == NSUB - 1)
            def _flush_tail():
                pltpu.sync_copy(
                    acc_sh.at[pl.ds(NSUB * ROWS_PER_SUB, TAIL_ROWS)],
                    out_hbm.at[pl.ds(NSUB * ROWS_PER_SUB, TAIL_ROWS)])


_sc_aggregate = pl.kernel(
    _sc_body,
    out_type=(jax.ShapeDtypeStruct((N_NODES, D), jnp.float32),
              jax.ShapeDtypeStruct((N_NODES, D), jnp.float32)),
    mesh=plsc.VectorSubcoreMesh(core_axis_name="c", subcore_axis_name="s"),
    scratch_types=[
        pltpu.VMEM((NPACK, 2 * CHUNK), jnp.int32),
        pltpu.VMEM((NPACK, 2 * CHUNK), jnp.int32),
        pltpu.VMEM((NPACK, 2 * CHUNK), jnp.float32),
        pltpu.VMEM((CHUNK, D), jnp.float32),
        pltpu.VMEM((CHUNK, D), jnp.float32),
        pltpu.VMEM_SHARED((N_NODES, D), jnp.float32),
        pltpu.SemaphoreType.DMA,
        pltpu.SemaphoreType.DMA,
        pltpu.SemaphoreType.DMA,
        pltpu.SemaphoreType.DMA,
    ],
)

ROW_BLK = 1000


def _tc_body(p0_ref, p1_ref, w_ref, o_ref):
    acc = p0_ref[...] + p1_ref[...]
    o_ref[...] = jnp.maximum(
        jnp.dot(acc, w_ref[...], preferred_element_type=jnp.float32), 0.0)


def _tc_finish(p0, p1, W):
    return pl.pallas_call(
        _tc_body,
        grid=(N_NODES // ROW_BLK,),
        in_specs=[
            pl.BlockSpec((ROW_BLK, D), lambda i: (i, 0)),
            pl.BlockSpec((ROW_BLK, D), lambda i: (i, 0)),
            pl.BlockSpec((D, D), lambda i: (0, 0)),
        ],
        out_specs=pl.BlockSpec((ROW_BLK, D), lambda i: (i, 0)),
        out_shape=jax.ShapeDtypeStruct((N_NODES, D), jnp.float32),
    )(p0, p1, W)


@jax.jit
def kernel(feat, edge_index, edge_weight, W):
    pad = E_PAD - N_EDGES
    pad_idx = (jnp.arange(pad, dtype=jnp.int32) * 13) % N_NODES
    src = jnp.concatenate(
        [edge_index[0], pad_idx]).reshape(NW, NPACK, 2 * CHUNK)
    dst = jnp.concatenate(
        [edge_index[1], pad_idx]).reshape(NW, NPACK, 2 * CHUNK)
    ew = jnp.concatenate(
        [edge_weight, jnp.zeros((pad,), jnp.float32)]
    ).reshape(NW, NPACK, 2 * CHUNK)
    p0, p1 = _sc_aggregate(feat, src, dst, ew)
    return _tc_finish(p0, p1, W)
